# baseline (speedup 1.0000x reference)
import functools

import jax
import jax.numpy as jnp
from jax import lax
from jax.experimental import pallas as pl
from jax.experimental.pallas import tpu as pltpu
from jax.experimental.pallas import tpu_sc as plsc

VOCAB = 1000
NUM_CORES = 2
NUM_SUBCORES = 16
NUM_WORKERS = NUM_CORES * NUM_SUBCORES
LANES = 16

ROWS_PER_CHUNK = 80


def _onehot_body(x_hbm, out_hbm, xv, buf, sem):
    wid = lax.axis_index("s") * NUM_CORES + lax.axis_index("c")
    rows_per_w = x_hbm.shape[0] // NUM_WORKERS
    chunks = rows_per_w // ROWS_PER_CHUNK
    base = wid * rows_per_w

    pltpu.sync_copy(x_hbm.at[pl.ds(base, rows_per_w)], xv)

    zeros16 = jnp.zeros((LANES,), jnp.float32)

    def zero_body(i, carry):
        for u in range(8):
            buf[pl.ds((i * 8 + u) * LANES, LANES)] = zeros16
        return carry

    lax.fori_loop(0, ROWS_PER_CHUNK * VOCAB // (8 * LANES), zero_body, 0)

    ones16 = jnp.full((LANES,), 1.0, jnp.float32)
    lane = lax.iota(jnp.int32, 16)

    def chunk_body(c, carry):
        for i in range(ROWS_PER_CHUNK // LANES):
            xs = xv[pl.ds(c * ROWS_PER_CHUNK + i * LANES, LANES)]
            flat = (lane + i * LANES) * VOCAB + xs
            plsc.store_scatter(buf, [flat], ones16)
        pltpu.async_copy(
            buf,
            out_hbm.at[pl.ds((base + c * ROWS_PER_CHUNK) * VOCAB,
                             ROWS_PER_CHUNK * VOCAB)],
            sem,
        ).wait()
        for i in range(ROWS_PER_CHUNK // LANES):
            xs = xv[pl.ds(c * ROWS_PER_CHUNK + i * LANES, LANES)]
            flat = (lane + i * LANES) * VOCAB + xs
            plsc.store_scatter(buf, [flat], zeros16)
        return carry

    lax.fori_loop(0, chunks, chunk_body, 0)


def kernel(x):
    B, L = x.shape
    n = B * L
    x_flat = x.reshape(n).astype(jnp.int32)

    run = functools.partial(
        pl.kernel,
        mesh=plsc.VectorSubcoreMesh(core_axis_name="c", subcore_axis_name="s"),
        out_type=jax.ShapeDtypeStruct((n * VOCAB,), jnp.float32),
        scratch_types=[
            pltpu.VMEM((n // NUM_WORKERS,), jnp.int32),
            pltpu.VMEM((ROWS_PER_CHUNK * VOCAB,), jnp.float32),
            pltpu.SemaphoreType.DMA,
        ],
        compiler_params=pltpu.CompilerParams(needs_layout_passes=False),
    )(_onehot_body)

    out = run(x_flat)
    return out.reshape(B, L, VOCAB)


# speedup vs baseline: 1.8916x; 1.8916x over previous
import functools

import jax
import jax.numpy as jnp
from jax import lax
from jax.experimental import pallas as pl
from jax.experimental.pallas import tpu as pltpu
from jax.experimental.pallas import tpu_sc as plsc

VOCAB = 1000
NUM_CORES = 2
NUM_SUBCORES = 16
NUM_WORKERS = NUM_CORES * NUM_SUBCORES
LANES = 16


def _onehot_body(x_hbm, out_hbm, xv, buf, sem):
    wid = lax.axis_index("s") * NUM_CORES + lax.axis_index("c")
    batches = out_hbm.shape[0]
    seq = out_hbm.shape[1]
    b_per_w = batches // NUM_WORKERS
    base_b = wid * b_per_w

    pltpu.sync_copy(x_hbm.at[pl.ds(base_b * seq, b_per_w * seq)], xv)

    zeros16 = jnp.zeros((LANES,), jnp.float32)
    ones16 = jnp.full((LANES,), 1.0, jnp.float32)
    lane = lax.iota(jnp.int32, LANES)

    def zero_body(l, carry):
        for j in range(seq * VOCAB // (LANES * seq)):
            buf[l, pl.ds(j * LANES, LANES)] = zeros16
        buf[l, pl.ds(VOCAB - LANES, LANES)] = zeros16
        return carry

    lax.fori_loop(0, seq, zero_body, 0)

    n_groups = (seq + LANES - 1) // LANES
    tail = seq - (n_groups - 1) * LANES

    def batch_body(b, carry):
        for g in range(n_groups):
            rows = lane + g * LANES
            cols = xv[pl.ds(b * seq + g * LANES, LANES)]
            mask = rows < seq if g == n_groups - 1 and tail != LANES else None
            plsc.store_scatter(buf, [rows, cols], ones16, mask=mask)
        pltpu.async_copy(buf, out_hbm.at[base_b + b], sem).wait()
        for g in range(n_groups):
            rows = lane + g * LANES
            cols = xv[pl.ds(b * seq + g * LANES, LANES)]
            mask = rows < seq if g == n_groups - 1 and tail != LANES else None
            plsc.store_scatter(buf, [rows, cols], zeros16, mask=mask)
        return carry

    lax.fori_loop(0, b_per_w, batch_body, 0)


def kernel(x):
    B, L = x.shape
    x_flat = x.reshape(B * L).astype(jnp.int32)

    run = functools.partial(
        pl.kernel,
        mesh=plsc.VectorSubcoreMesh(core_axis_name="c", subcore_axis_name="s"),
        out_type=jax.ShapeDtypeStruct((B, L, VOCAB), jnp.float32),
        scratch_types=[
            pltpu.VMEM((B * L // NUM_WORKERS,), jnp.int32),
            pltpu.VMEM((L, VOCAB), jnp.float32),
            pltpu.SemaphoreType.DMA,
        ],
        compiler_params=pltpu.CompilerParams(needs_layout_passes=False),
    )(_onehot_body)

    return run(x_flat)
